# baseline (speedup 1.0000x reference)
import functools

import jax
import jax.numpy as jnp
from jax import lax
from jax.experimental import pallas as pl
from jax.experimental.pallas import tpu as pltpu
from jax.experimental.pallas import tpu_sc as plsc

NC = 2
NS = 16
NW = NC * NS
LANES = 16
HH = 112
CPR = 6


def _mixup_body(rows_per, x_hbm, bidx_hbm, w_hbm, out_hbm,
                bidx_v, w_v, a0, b0, a1, b1, si0, si1, so0, so1):
    per_w = rows_per * CPR
    c = lax.axis_index("c")
    s = lax.axis_index("s")
    wid = s * NC + c
    row0 = wid * rows_per

    pltpu.sync_copy(bidx_hbm.at[pl.ds(row0, rows_per), :], bidx_v)
    pltpu.sync_copy(w_hbm, w_v)
    wa = w_v[...]
    wb = 1.0 - wa

    abufs = (a0, a1)
    bbufs = (b0, b1)
    isems = (si0, si1)
    osems = (so0, so1)

    def start_in(t, k):
        r = t // CPR
        q = t - r * CPR
        ci = q // 2
        h = (q - ci * 2) * HH
        arow = row0 + r
        brow = bidx_v[r, :][0]
        pltpu.async_copy(
            x_hbm.at[arow, ci, pl.ds(h, HH), :], abufs[k], isems[k])
        pltpu.async_copy(
            x_hbm.at[brow, ci, pl.ds(h, HH), :], bbufs[k], isems[k])

    def wait_in(k):
        pltpu.make_async_copy(
            x_hbm.at[0, 0, pl.ds(0, HH), :], abufs[k], isems[k]).wait()
        pltpu.make_async_copy(
            x_hbm.at[0, 0, pl.ds(0, HH), :], bbufs[k], isems[k]).wait()

    def start_out(t, k):
        r = t // CPR
        q = t - r * CPR
        ci = q // 2
        h = (q - ci * 2) * HH
        pltpu.async_copy(
            abufs[k], out_hbm.at[row0 + r, ci, pl.ds(h, HH), :], osems[k])

    def wait_out(k):
        pltpu.make_async_copy(
            abufs[k], out_hbm.at[0, 0, pl.ds(0, HH), :], osems[k]).wait()

    start_in(0, 0)

    def group(g, _):
        for k in (0, 1):
            t = g * 2 + k
            nk = 1 - k

            @pl.when(t + 1 < per_w)
            def _():
                @pl.when(t >= 1)
                def _():
                    wait_out(nk)
                start_in(t + 1, nk)

            wait_in(k)

            def vec_body(j, _, k=k):
                for u in range(W_VREGS):
                    sl = pl.ds(u * LANES, LANES)
                    abufs[k][j, sl] = (wa * abufs[k][j, sl]
                                       + wb * bbufs[k][j, sl])
                return 0

            lax.fori_loop(0, HH, vec_body, 0)
            start_out(t, k)
        return 0

    lax.fori_loop(0, per_w // 2, group, 0)
    wait_out(0)
    wait_out(1)


W_VREGS = 14


def kernel(inputs, index, lam):
    B, C, H, W = inputs.shape
    assert H % HH == 0 and C * (H // HH) == CPR and W == W_VREGS * LANES
    assert B % NW == 0
    rows_per = B // NW

    idx = index.astype(jnp.int32)
    bidx = idx.reshape(B, 1) * jnp.ones((1, LANES), jnp.int32)
    w = jnp.full((LANES,), lam, dtype=jnp.float32)

    mesh = plsc.VectorSubcoreMesh(
        core_axis_name="c", subcore_axis_name="s",
        num_cores=NC, num_subcores=NS)

    run = pl.kernel(
        functools.partial(_mixup_body, rows_per),
        out_type=jax.ShapeDtypeStruct((B, C, H, W), jnp.float32),
        mesh=mesh,
        scratch_types=[
            pltpu.VMEM((rows_per, LANES), jnp.int32),
            pltpu.VMEM((LANES,), jnp.float32),
            pltpu.VMEM((HH, W), jnp.float32),
            pltpu.VMEM((HH, W), jnp.float32),
            pltpu.VMEM((HH, W), jnp.float32),
            pltpu.VMEM((HH, W), jnp.float32),
            pltpu.SemaphoreType.DMA,
            pltpu.SemaphoreType.DMA,
            pltpu.SemaphoreType.DMA,
            pltpu.SemaphoreType.DMA,
        ],
    )
    return run(inputs, bidx, w)
